# baseline (speedup 1.0000x reference)
import functools

import jax
import jax.numpy as jnp
from jax import lax
from jax.experimental import pallas as pl
from jax.experimental.pallas import tpu as pltpu
from jax.experimental.pallas import tpu_sc as plsc

NC, NS, L = 2, 16, 16
NW = NC * NS
B, SEQ, C = 4096, 200, 64
BPW = B // NW
NBLK = -(-SEQ // L)
LPAD = NBLK * L
HALF = SEQ // 2


def _make_kernel():
    mesh = plsc.VectorSubcoreMesh(
        core_axis_name="c", subcore_axis_name="s",
        num_cores=NC, num_subcores=NS)

    @functools.partial(
        pl.kernel,
        out_type=jax.ShapeDtypeStruct((B, C, SEQ), jnp.float32),
        mesh=mesh,
        scratch_types=[
            pltpu.VMEM((2, HALF), jnp.int32),
            pltpu.VMEM((LPAD,), jnp.float32),
            pltpu.VMEM((LPAD, C), jnp.float32),
            pltpu.VMEM((C, LPAD), jnp.float32),
            pltpu.SemaphoreType.DMA,
        ],
        compiler_params=pltpu.CompilerParams(use_tc_tiling_on_sc=False, needs_layout_passes=False),
    )
    def k(x_hbm, m_hbm, t_hbm, out_hbm, idx_v, mask_v, rows_v, outt_v, sem):
        wid = lax.axis_index("s") * NC + lax.axis_index("c")
        iota = lax.iota(jnp.int32, L)

        def body(i, carry):
            b = wid * BPW + i
            pltpu.sync_copy(x_hbm.at[b], idx_v)
            pltpu.sync_copy(m_hbm.at[b], mask_v.at[pl.ds(0, SEQ)])
            cp0 = pltpu.async_copy(
                t_hbm.at[idx_v.at[0]], rows_v.at[pl.ds(0, HALF)], sem)
            cp1 = pltpu.async_copy(
                t_hbm.at[idx_v.at[1]], rows_v.at[pl.ds(HALF, HALF)], sem)
            cp0.wait()
            cp1.wait()

            mvs = [mask_v[pl.ds(j * L, L)] for j in range(NBLK)]
            lidx = [iota + (j * L) for j in range(NBLK)]

            def cbody(c, carry2):
                csplat = jnp.full((L,), c, jnp.int32)
                for j in range(NBLK):
                    v = plsc.load_gather(rows_v, [lidx[j], csplat])
                    outt_v[c, pl.ds(j * L, L)] = v * mvs[j]
                return carry2

            lax.fori_loop(0, C, cbody, 0)
            pltpu.sync_copy(outt_v.at[:, pl.ds(0, SEQ)], out_hbm.at[b])
            return carry

        lax.fori_loop(0, BPW, body, 0)

    return k


_sc_kernel = _make_kernel()


@jax.jit
def kernel(x, mask, table):
    x2 = x.astype(jnp.int32).reshape(B, 2, HALF)
    m2 = mask.reshape(B, SEQ)
    return _sc_kernel(x2, m2, table)


# speedup vs baseline: 1.1393x; 1.1393x over previous
import functools

import jax
import jax.numpy as jnp
from jax import lax
from jax.experimental import pallas as pl
from jax.experimental.pallas import tpu as pltpu
from jax.experimental.pallas import tpu_sc as plsc

NC, NS, L = 2, 16, 16
NW = NC * NS
B, SEQ, C = 4096, 200, 64
BPW = B // NW
HALF = SEQ // 2
OFFS = [j * L for j in range(SEQ // L)] + [SEQ - L]
NBLK = len(OFFS)


def _make_kernel():
    mesh = plsc.VectorSubcoreMesh(
        core_axis_name="c", subcore_axis_name="s",
        num_cores=NC, num_subcores=NS)

    @functools.partial(
        pl.kernel,
        out_type=jax.ShapeDtypeStruct((B, C, SEQ), jnp.float32),
        mesh=mesh,
        scratch_types=[
            pltpu.VMEM((2, 2, HALF), jnp.int32),
            pltpu.VMEM((2, SEQ), jnp.float32),
            pltpu.VMEM((2, SEQ, C), jnp.float32),
            pltpu.VMEM((2, C, SEQ), jnp.float32),
            pltpu.SemaphoreType.DMA((2,)),
            pltpu.SemaphoreType.DMA((2,)),
            pltpu.SemaphoreType.DMA((2,)),
        ],
        compiler_params=pltpu.CompilerParams(
            use_tc_tiling_on_sc=False, needs_layout_passes=False),
    )
    def k(x_hbm, m_hbm, t_hbm, out_hbm, idx_v, mask_v, rows_v, outt_v,
          in_sem, gat_sem, out_sem):
        wid = lax.axis_index("s") * NC + lax.axis_index("c")
        b0 = wid * BPW
        iota = lax.iota(jnp.int32, L)
        lidx = [iota + o for o in OFFS]

        def start_in(i, p):
            pltpu.async_copy(x_hbm.at[b0 + i], idx_v.at[p], in_sem.at[p])
            pltpu.async_copy(m_hbm.at[b0 + i], mask_v.at[p], in_sem.at[p])

        def wait_in(p):
            pltpu.make_async_copy(
                x_hbm.at[0], idx_v.at[p], in_sem.at[p]).wait()
            pltpu.make_async_copy(
                m_hbm.at[0], mask_v.at[p], in_sem.at[p]).wait()

        def start_gather(p):
            pltpu.async_copy(
                t_hbm.at[idx_v.at[p].at[0]],
                rows_v.at[p].at[pl.ds(0, HALF)], gat_sem.at[p])
            pltpu.async_copy(
                t_hbm.at[idx_v.at[p].at[1]],
                rows_v.at[p].at[pl.ds(HALF, HALF)], gat_sem.at[p])

        def wait_gather(p):
            pltpu.make_async_copy(
                t_hbm.at[pl.ds(0, SEQ)], rows_v.at[p], gat_sem.at[p]).wait()

        def start_out(i, p):
            pltpu.async_copy(outt_v.at[p], out_hbm.at[b0 + i], out_sem.at[p])

        def wait_out(p):
            pltpu.make_async_copy(
                outt_v.at[p], out_hbm.at[0], out_sem.at[p]).wait()

        start_in(0, 0)
        start_in(1, 1)
        wait_in(0)
        start_gather(0)

        def body(i, carry):
            p = lax.rem(i, 2)
            q = 1 - p

            @pl.when(i < BPW - 1)
            def _():
                wait_in(q)
                start_gather(q)

            wait_gather(p)
            mvs = [mask_v[p, pl.ds(o, L)] for o in OFFS]

            @pl.when(i < BPW - 2)
            def _():
                start_in(i + 2, p)

            @pl.when(i >= 2)
            def _():
                wait_out(p)

            rows = rows_v.at[p]

            def cbody(c, carry2):
                csplat = jnp.full((L,), c, jnp.int32)
                for j in range(NBLK):
                    v = plsc.load_gather(rows, [lidx[j], csplat])
                    outt_v[p, c, pl.ds(OFFS[j], L)] = v * mvs[j]
                return carry2

            lax.fori_loop(0, C, cbody, 0)
            start_out(i, p)
            return carry

        lax.fori_loop(0, BPW, body, 0)
        wait_out(0)
        wait_out(1)

    return k


_sc_kernel = _make_kernel()


@jax.jit
def kernel(x, mask, table):
    x2 = x.astype(jnp.int32).reshape(B, 2, HALF)
    m2 = mask.reshape(B, SEQ)
    return _sc_kernel(x2, m2, table)


# speedup vs baseline: 1.5781x; 1.3851x over previous
import functools

import jax
import jax.numpy as jnp
from jax import lax
from jax.experimental import pallas as pl
from jax.experimental.pallas import tpu as pltpu
from jax.experimental.pallas import tpu_sc as plsc

NC, NS, L = 2, 16, 16
NW = NC * NS
B, SEQ, C = 4096, 200, 64
BPW = B // NW
HALF = SEQ // 2
OFFS = [j * L for j in range(SEQ // L)] + [SEQ - L]
NBLK = len(OFFS)


def _make_kernel():
    mesh = plsc.VectorSubcoreMesh(
        core_axis_name="c", subcore_axis_name="s",
        num_cores=NC, num_subcores=NS)

    @functools.partial(
        pl.kernel,
        out_type=jax.ShapeDtypeStruct((B, C, SEQ), jnp.float32),
        mesh=mesh,
        scratch_types=[
            pltpu.VMEM((2, 2, HALF), jnp.int32),
            pltpu.VMEM((2, SEQ), jnp.float32),
            pltpu.VMEM((2, SEQ, C), jnp.float32),
            pltpu.VMEM((2, C, SEQ), jnp.float32),
            pltpu.SemaphoreType.DMA((2,)),
            pltpu.SemaphoreType.DMA((2,)),
            pltpu.SemaphoreType.DMA((2,)),
        ],
        compiler_params=pltpu.CompilerParams(
            use_tc_tiling_on_sc=False, needs_layout_passes=False),
    )
    def k(x_hbm, m_hbm, t_hbm, out_hbm, idx_v, mask_v, rows_v, outt_v,
          in_sem, gat_sem, out_sem):
        wid = lax.axis_index("s") * NC + lax.axis_index("c")
        b0 = wid * BPW
        iota = lax.iota(jnp.int32, L)
        lidx = [iota + o for o in OFFS]

        def start_in(i, p):
            pltpu.async_copy(x_hbm.at[b0 + i], idx_v.at[p], in_sem.at[p])
            pltpu.async_copy(m_hbm.at[b0 + i], mask_v.at[p], in_sem.at[p])

        def wait_in(p):
            pltpu.make_async_copy(
                x_hbm.at[0], idx_v.at[p], in_sem.at[p]).wait()
            pltpu.make_async_copy(
                m_hbm.at[0], mask_v.at[p], in_sem.at[p]).wait()

        def start_gather(p):
            pltpu.async_copy(
                t_hbm.at[idx_v.at[p].at[0]],
                rows_v.at[p].at[pl.ds(0, HALF)], gat_sem.at[p])
            pltpu.async_copy(
                t_hbm.at[idx_v.at[p].at[1]],
                rows_v.at[p].at[pl.ds(HALF, HALF)], gat_sem.at[p])

        def wait_gather(p):
            pltpu.make_async_copy(
                t_hbm.at[pl.ds(0, SEQ)], rows_v.at[p], gat_sem.at[p]).wait()

        def start_out(i, p):
            pltpu.async_copy(outt_v.at[p], out_hbm.at[b0 + i], out_sem.at[p])

        def wait_out(p):
            pltpu.make_async_copy(
                outt_v.at[p], out_hbm.at[0], out_sem.at[p]).wait()

        start_in(0, 0)
        start_in(1, 1)
        wait_in(0)
        start_gather(0)

        def body(i, carry):
            p = lax.rem(i, 2)
            q = 1 - p

            @pl.when(i < BPW - 1)
            def _():
                wait_in(q)
                start_gather(q)

            wait_gather(p)
            mvs = [mask_v[p, pl.ds(o, L)] for o in OFFS]

            @pl.when(i < BPW - 2)
            def _():
                start_in(i + 2, p)

            @pl.when(i >= 2)
            def _():
                wait_out(p)

            rows = rows_v.at[p]

            @plsc.parallel_loop(0, C, 1, unroll=4)
            def cbody(c):
                csplat = jnp.full((L,), c, jnp.int32)
                for j in range(NBLK):
                    v = plsc.load_gather(rows, [lidx[j], csplat])
                    outt_v[p, c, pl.ds(OFFS[j], L)] = v * mvs[j]
            start_out(i, p)
            return carry

        lax.fori_loop(0, BPW, body, 0)
        wait_out(0)
        wait_out(1)

    return k


_sc_kernel = _make_kernel()


@jax.jit
def kernel(x, mask, table):
    x2 = x.astype(jnp.int32).reshape(B, 2, HALF)
    m2 = mask.reshape(B, SEQ)
    return _sc_kernel(x2, m2, table)


# speedup vs baseline: 1.5786x; 1.0003x over previous
import functools

import jax
import jax.numpy as jnp
from jax import lax
from jax.experimental import pallas as pl
from jax.experimental.pallas import tpu as pltpu
from jax.experimental.pallas import tpu_sc as plsc

NC, NS, L = 2, 16, 16
NW = NC * NS
B, SEQ, C = 4096, 200, 64
BPW = B // NW
CH0, CH1 = 104, 96
OFFS = [j * L for j in range(SEQ // L)] + [SEQ - L]
NBLK = len(OFFS)


def _make_kernel():
    mesh = plsc.VectorSubcoreMesh(
        core_axis_name="c", subcore_axis_name="s",
        num_cores=NC, num_subcores=NS)

    @functools.partial(
        pl.kernel,
        out_type=jax.ShapeDtypeStruct((B, C, SEQ), jnp.float32),
        mesh=mesh,
        scratch_types=[
            pltpu.VMEM((2, SEQ), jnp.int32),
            pltpu.VMEM((2, SEQ), jnp.float32),
            pltpu.VMEM((2, SEQ, C), jnp.float32),
            pltpu.VMEM((2, C, SEQ), jnp.float32),
            pltpu.SemaphoreType.DMA((2,)),
            pltpu.SemaphoreType.DMA((2,)),
            pltpu.SemaphoreType.DMA((2,)),
        ],
        compiler_params=pltpu.CompilerParams(
            use_tc_tiling_on_sc=False, needs_layout_passes=False),
    )
    def k(x_hbm, m_hbm, t_hbm, out_hbm, idx_v, mask_v, rows_v, outt_v,
          in_sem, gat_sem, out_sem):
        wid = lax.axis_index("s") * NC + lax.axis_index("c")
        b0 = wid * BPW
        iota = lax.iota(jnp.int32, L)
        lidx = [iota + o for o in OFFS]

        def start_in(i, p):
            pltpu.async_copy(x_hbm.at[b0 + i], idx_v.at[p], in_sem.at[p])
            pltpu.async_copy(m_hbm.at[b0 + i, 0], mask_v.at[p], in_sem.at[p])

        def wait_in(p):
            pltpu.make_async_copy(
                x_hbm.at[0], idx_v.at[p], in_sem.at[p]).wait()
            pltpu.make_async_copy(
                m_hbm.at[0, 0], mask_v.at[p], in_sem.at[p]).wait()

        def start_gather(p):
            pltpu.async_copy(
                t_hbm.at[idx_v.at[p].at[pl.ds(0, CH0)]],
                rows_v.at[p].at[pl.ds(0, CH0)], gat_sem.at[p])
            pltpu.async_copy(
                t_hbm.at[idx_v.at[p].at[pl.ds(CH0, CH1)]],
                rows_v.at[p].at[pl.ds(CH0, CH1)], gat_sem.at[p])

        def wait_gather(p):
            pltpu.make_async_copy(
                t_hbm.at[pl.ds(0, SEQ)], rows_v.at[p], gat_sem.at[p]).wait()

        def start_out(i, p):
            pltpu.async_copy(outt_v.at[p], out_hbm.at[b0 + i], out_sem.at[p])

        def wait_out(p):
            pltpu.make_async_copy(
                outt_v.at[p], out_hbm.at[0], out_sem.at[p]).wait()

        start_in(0, 0)
        start_in(1, 1)
        wait_in(0)
        start_gather(0)

        def body(i, carry):
            p = lax.rem(i, 2)
            q = 1 - p

            @pl.when(i < BPW - 1)
            def _():
                wait_in(q)
                start_gather(q)

            wait_gather(p)
            mvs = [mask_v[p, pl.ds(o, L)] for o in OFFS]

            @pl.when(i < BPW - 2)
            def _():
                start_in(i + 2, p)

            @pl.when(i >= 2)
            def _():
                wait_out(p)

            rows = rows_v.at[p]

            @plsc.parallel_loop(0, C, 1, unroll=4)
            def cbody(c):
                csplat = jnp.full((L,), c, jnp.int32)
                for j in range(NBLK):
                    v = plsc.load_gather(rows, [lidx[j], csplat])
                    outt_v[p, c, pl.ds(OFFS[j], L)] = v * mvs[j]

            start_out(i, p)
            return carry

        lax.fori_loop(0, BPW, body, 0)
        wait_out(0)
        wait_out(1)

    return k


_sc_kernel = _make_kernel()


@jax.jit
def kernel(x, mask, table):
    return _sc_kernel(x.astype(jnp.int32), mask, table)


# speedup vs baseline: 1.9132x; 1.2120x over previous
import functools

import jax
import jax.numpy as jnp
from jax import lax
from jax.experimental import pallas as pl
from jax.experimental.pallas import tpu as pltpu
from jax.experimental.pallas import tpu_sc as plsc

NC, NS, L = 2, 16, 16
NW = NC * NS
B, SEQ, C = 4096, 200, 64
V = 1_000_000
RCH = 128
NCHUNK = V // RCH
TAIL = V - NCHUNK * RCH
BPW = B // NW
NOCT = SEQ // 8


def _mesh():
    return plsc.VectorSubcoreMesh(
        core_axis_name="c", subcore_axis_name="s",
        num_cores=NC, num_subcores=NS)


def _make_k1():
    @functools.partial(
        pl.kernel,
        out_type=jax.ShapeDtypeStruct((V, 128), jnp.float32),
        mesh=_mesh(),
        scratch_types=[
            pltpu.VMEM((2, C, RCH), jnp.float32),
            pltpu.VMEM((2, RCH, 128), jnp.float32),
            pltpu.VMEM((C, TAIL), jnp.float32),
            pltpu.VMEM((TAIL, 128), jnp.float32),
            pltpu.SemaphoreType.DMA((2,)),
            pltpu.SemaphoreType.DMA((2,)),
        ],
        compiler_params=pltpu.CompilerParams(
            use_tc_tiling_on_sc=True, needs_layout_passes=False),
    )
    def k1(tt_hbm, tail_hbm, trm_hbm, tbuf, obuf, tailin, tailout,
           in_sem, out_sem):
        wid = lax.axis_index("s") * NC + lax.axis_index("c")
        niter = (NCHUNK - wid + NW - 1) // NW
        ciota = lax.iota(jnp.int32, L)

        def r0_of(i):
            return pl.multiple_of((wid + i * NW) * RCH, RCH)

        def start_in(i, p):
            pltpu.async_copy(
                tt_hbm.at[:, pl.ds(r0_of(i), RCH)], tbuf.at[p], in_sem.at[p])

        def wait_in(p):
            pltpu.make_async_copy(
                tt_hbm.at[:, pl.ds(0, RCH)], tbuf.at[p], in_sem.at[p]).wait()

        def start_out(i, p):
            pltpu.async_copy(
                obuf.at[p], trm_hbm.at[pl.ds(r0_of(i), RCH)], out_sem.at[p])

        def wait_out(p):
            pltpu.make_async_copy(
                obuf.at[p], trm_hbm.at[pl.ds(0, RCH)], out_sem.at[p]).wait()

        @pl.when(wid == 0)
        def _():
            pltpu.sync_copy(tail_hbm, tailin)

            @plsc.parallel_loop(0, TAIL, 1, unroll=4)
            def tbody(r):
                rsplat = jnp.full((L,), r, jnp.int32)
                for k in range(C // L):
                    v = plsc.load_gather(tailin, [ciota + k * L, rsplat])
                    tailout[r, pl.ds(k * L, L)] = v

            pltpu.sync_copy(tailout, trm_hbm.at[pl.ds(V - TAIL, TAIL)])

        start_in(0, 0)

        def body(i, carry):
            p = lax.rem(i, 2)

            @pl.when(i + 1 < niter)
            def _():
                start_in(i + 1, 1 - p)

            wait_in(p)

            @pl.when(i >= 2)
            def _():
                wait_out(p)

            tb = tbuf.at[p]

            @plsc.parallel_loop(0, RCH, 1, unroll=4)
            def rbody(r):
                rsplat = jnp.full((L,), r, jnp.int32)
                for k in range(C // L):
                    v = plsc.load_gather(tb, [ciota + k * L, rsplat])
                    obuf[p, r, pl.ds(k * L, L)] = v

            start_out(i, p)
            return carry

        lax.fori_loop(0, niter, body, 0)
        wait_out(lax.rem(niter - 1, 2))
        wait_out(lax.rem(niter, 2))

    return k1


def _make_k2():
    @functools.partial(
        pl.kernel,
        out_type=jax.ShapeDtypeStruct((C, SEQ, B), jnp.float32),
        mesh=_mesh(),
        scratch_types=[
            pltpu.VMEM((2, 8, 128), jnp.int32),
            pltpu.VMEM((2, 8, 128), jnp.float32),
            pltpu.VMEM((128,), jnp.int32),
            pltpu.VMEM((2, 128, 128), jnp.float32),
            pltpu.VMEM((C, 8, 128), jnp.float32),
            pltpu.SemaphoreType.DMA((2,)),
            pltpu.SemaphoreType.DMA((2,)),
            pltpu.SemaphoreType.DMA,
        ],
        compiler_params=pltpu.CompilerParams(
            use_tc_tiling_on_sc=True, needs_layout_passes=False),
    )
    def k2(xt_hbm, mt_hbm, trm_hbm, out_hbm,
           idx_v, mask_v, idx1d, rows_v, outt_v, in_sem, gat_sem, out_sem):
        wid = lax.axis_index("s") * NC + lax.axis_index("c")
        b0 = pl.multiple_of(wid * BPW, 128)
        biota = lax.iota(jnp.int32, L)

        def l0_of(j):
            return pl.multiple_of(j * 8, 8)

        def start_in(j, p):
            pltpu.async_copy(
                xt_hbm.at[pl.ds(l0_of(j), 8), pl.ds(b0, 128)], idx_v.at[p],
                in_sem.at[p])
            pltpu.async_copy(
                mt_hbm.at[pl.ds(l0_of(j), 8), pl.ds(b0, 128)], mask_v.at[p],
                in_sem.at[p])

        def wait_in(p):
            pltpu.make_async_copy(
                xt_hbm.at[pl.ds(0, 8), pl.ds(0, 128)], idx_v.at[p],
                in_sem.at[p]).wait()
            pltpu.make_async_copy(
                mt_hbm.at[pl.ds(0, 8), pl.ds(0, 128)], mask_v.at[p],
                in_sem.at[p]).wait()

        def build_idx1d(pj, dl):
            for k in range(128 // L):
                idx1d[pl.ds(k * L, L)] = idx_v[pj, dl, pl.ds(k * L, L)]

        def start_gather(pr):
            pltpu.async_copy(
                trm_hbm.at[idx1d], rows_v.at[pr], gat_sem.at[pr])

        def wait_gather(pr):
            pltpu.make_async_copy(
                trm_hbm.at[pl.ds(0, 128)], rows_v.at[pr],
                gat_sem.at[pr]).wait()

        def wait_out():
            pltpu.make_async_copy(
                outt_v, out_hbm.at[:, pl.ds(0, 8), pl.ds(0, 128)],
                out_sem).wait()

        start_in(0, 0)
        start_in(1, 1)
        wait_in(0)
        build_idx1d(0, 0)
        start_gather(0)

        def octet(j, carry):
            pj = lax.rem(j, 2)

            def lbody(dl, carry2):
                pr = lax.rem(dl, 2)
                wait_gather(pr)

                @pl.when(dl < 7)
                def _():
                    build_idx1d(pj, dl + 1)
                    start_gather(1 - pr)

                @pl.when((dl == 7) & (j < NOCT - 1))
                def _():
                    wait_in(1 - pj)
                    build_idx1d(1 - pj, 0)
                    start_gather(1 - pr)
                rows = rows_v.at[pr]
                mvs = [mask_v[pj, dl, pl.ds(k * L, L)] for k in range(8)]
                lidx = [biota + k * L for k in range(8)]

                @pl.when((j > 0) & (dl == 0))
                def _():
                    wait_out()

                @plsc.parallel_loop(0, C, 1, unroll=2)
                def cbody(c):
                    csplat = jnp.full((L,), c, jnp.int32)
                    for k in range(8):
                        v = plsc.load_gather(rows, [lidx[k], csplat])
                        outt_v[c, dl, pl.ds(k * L, L)] = v * mvs[k]

                return carry2

            lax.fori_loop(0, 8, lbody, 0)

            @pl.when(j < NOCT - 2)
            def _():
                start_in(j + 2, pj)

            pltpu.async_copy(
                outt_v,
                out_hbm.at[:, pl.ds(l0_of(j), 8), pl.ds(b0, 128)], out_sem)
            return carry

        lax.fori_loop(0, NOCT, octet, 0)
        wait_out()

    return k2


_k1 = _make_k1()
_k2 = _make_k2()


@jax.jit
def kernel(x, mask, table):
    xt = x.astype(jnp.int32).T
    mt = mask.reshape(B, SEQ).T
    tt = table.T
    ttail = lax.slice(table, (V - TAIL, 0), (V, C)).T
    trm = _k1(tt, ttail)
    outt = _k2(xt, mt, trm)
    return outt.transpose(2, 0, 1)
